# Initial kernel scaffold; baseline (speedup 1.0000x reference)
import jax
import jax.numpy as jnp
from jax.experimental import pallas as pl


def kernel(x, W1, b1, W2, b2):
    raise NotImplementedError("write your pallas kernel here")


# speedup vs baseline: 450.8526x; 450.8526x over previous
import jax
import jax.numpy as jnp
import numpy as np
from jax.experimental import pallas as pl

_L = 1000


def _build_candidates():
    one = np.float32(1.0)
    thr = np.float32(0.5)
    traj = np.empty(_L, np.float32)
    traj[0] = np.float32(0.5)
    for i in range(1, _L):
        xp = traj[i - 1]
        traj[i] = xp / thr if xp < thr else (one - xp) / (one - thr)
    z = np.zeros(1, np.float32)
    cs_gt = np.concatenate(
        [z, np.cumsum((traj > 0.5).astype(np.float32), dtype=np.float32)])
    cs_sq = np.concatenate([z, np.cumsum(traj * traj, dtype=np.float32)])
    cs_ent = np.concatenate(
        [z, np.cumsum(traj * np.log2(traj + np.float32(1e-10)),
                      dtype=np.float32)])
    _, first = np.unique(traj, return_index=True)
    cands = []
    for i in np.sort(first):
        i = int(i)
        cands.append((float(traj[i]), float(np.float32(i)),
                      float(cs_gt[i]), float(cs_sq[i]), float(cs_ent[i])))
    return cands


_CANDS = _build_candidates()


def _fused_kernel(x_ref, w1_ref, b1_ref, w2_ref, b2_ref, o_ref):
    v = x_ref[...]
    u0, i0, g0, s0, e0 = _CANDS[0]
    best = jnp.abs(v - u0)
    f_idx = jnp.full_like(v, i0)
    f_gt = jnp.full_like(v, g0)
    f_sq = jnp.full_like(v, s0)
    f_ent = jnp.full_like(v, e0)
    for (u, i, g, s, e) in _CANDS[1:]:
        d = jnp.abs(v - u)
        m = d < best
        best = jnp.where(m, d, best)
        f_idx = jnp.where(m, i, f_idx)
        f_gt = jnp.where(m, g, f_gt)
        f_sq = jnp.where(m, s, f_sq)
        f_ent = jnp.where(m, e, f_ent)
    ttss = f_gt / f_idx
    feats = jnp.concatenate([ttss, f_sq, f_idx, -f_ent], axis=1)
    h = jnp.dot(feats, w1_ref[...], preferred_element_type=jnp.float32)
    h = jnp.maximum(h + b1_ref[...], 0.0)
    o_ref[...] = (jnp.dot(h, w2_ref[...], preferred_element_type=jnp.float32)
                  + b2_ref[...])


def kernel(x, W1, b1, W2, b2):
    B, D = x.shape
    H = W1.shape[0]
    O = W2.shape[0]
    w1t = W1.T.reshape(D, 4, H).transpose(1, 0, 2).reshape(4 * D, H)
    w2t = W2.T
    TB = 512
    out = pl.pallas_call(
        _fused_kernel,
        grid=(B // TB,),
        in_specs=[
            pl.BlockSpec((TB, D), lambda i: (i, 0)),
            pl.BlockSpec((4 * D, H), lambda i: (0, 0)),
            pl.BlockSpec((1, H), lambda i: (0, 0)),
            pl.BlockSpec((H, O), lambda i: (0, 0)),
            pl.BlockSpec((1, O), lambda i: (0, 0)),
        ],
        out_specs=pl.BlockSpec((TB, O), lambda i: (i, 0)),
        out_shape=jax.ShapeDtypeStruct((B, O), jnp.float32),
    )(x, w1t, b1.reshape(1, H), w2t, b2.reshape(1, O))
    return out


# speedup vs baseline: 555.2223x; 1.2315x over previous
import jax
import jax.numpy as jnp
import numpy as np
from jax.experimental import pallas as pl
from jax.experimental.pallas import tpu as pltpu

_L = 1000


def _build_candidates():
    one = np.float32(1.0)
    thr = np.float32(0.5)
    traj = np.empty(_L, np.float32)
    traj[0] = np.float32(0.5)
    for i in range(1, _L):
        xp = traj[i - 1]
        traj[i] = xp / thr if xp < thr else (one - xp) / (one - thr)
    z = np.zeros(1, np.float32)
    cs_gt = np.concatenate(
        [z, np.cumsum((traj > 0.5).astype(np.float32), dtype=np.float32)])
    cs_sq = np.concatenate([z, np.cumsum(traj * traj, dtype=np.float32)])
    cs_ent = np.concatenate(
        [z, np.cumsum(traj * np.log2(traj + np.float32(1e-10)),
                      dtype=np.float32)])
    _, first = np.unique(traj, return_index=True)
    cands = []
    for i in np.sort(first):
        i = int(i)
        cands.append((float(traj[i]), float(np.float32(i)),
                      float(cs_gt[i]), float(cs_sq[i]), float(cs_ent[i])))
    return cands


_CANDS = _build_candidates()


def _fused_kernel(x_ref, w1_ref, b1_ref, w2_ref, b2_ref, o_ref):
    v = x_ref[...]
    u0, i0, g0, s0, e0 = _CANDS[0]
    best = jnp.abs(v - u0)
    f_idx = jnp.full_like(v, i0)
    f_gt = jnp.full_like(v, g0)
    f_sq = jnp.full_like(v, s0)
    f_ent = jnp.full_like(v, e0)
    for (u, i, g, s, e) in _CANDS[1:]:
        d = jnp.abs(v - u)
        m = d < best
        best = jnp.where(m, d, best)
        f_idx = jnp.where(m, i, f_idx)
        f_gt = jnp.where(m, g, f_gt)
        f_sq = jnp.where(m, s, f_sq)
        f_ent = jnp.where(m, e, f_ent)
    TB, D = v.shape
    blocked = jnp.concatenate([f_gt, f_sq, f_idx, -f_ent], axis=1)
    r = jax.lax.broadcasted_iota(jnp.int32, (4 * D, 4 * D), 0)
    j = jax.lax.broadcasted_iota(jnp.int32, (4 * D, 4 * D), 1)
    perm = (j == 4 * (r % D) + r // D).astype(jnp.float32)
    feats0 = jax.lax.dot_general(blocked, perm, (((1,), (0,)), ((), ())),
                                 preferred_element_type=jnp.float32)
    rd = jax.lax.broadcasted_iota(jnp.int32, (D, 4 * D), 0)
    jd = jax.lax.broadcasted_iota(jnp.int32, (D, 4 * D), 1)
    sel0 = (jd == 4 * rd).astype(jnp.float32)
    idx0 = jax.lax.dot_general(f_idx, sel0, (((1,), (0,)), ((), ())),
                               preferred_element_type=jnp.float32)
    lane_c = jax.lax.broadcasted_iota(jnp.int32, (TB, 4 * D), 1) % 4
    feats = jnp.where(lane_c == 0, feats0 / idx0, feats0)
    h = jax.lax.dot_general(feats, w1_ref[...], (((1,), (1,)), ((), ())),
                            preferred_element_type=jnp.float32)
    h = jnp.maximum(h + b1_ref[...], 0.0)
    o_ref[...] = (jax.lax.dot_general(h, w2_ref[...], (((1,), (1,)), ((), ())),
                                      preferred_element_type=jnp.float32)
                  + b2_ref[...])


def kernel(x, W1, b1, W2, b2):
    B, D = x.shape
    H = W1.shape[0]
    O = W2.shape[0]
    TB = 512
    out = pl.pallas_call(
        _fused_kernel,
        grid=(B // TB,),
        in_specs=[
            pl.BlockSpec((TB, D), lambda i: (i, 0)),
            pl.BlockSpec((H, 4 * D), lambda i: (0, 0)),
            pl.BlockSpec((1, H), lambda i: (0, 0)),
            pl.BlockSpec((O, H), lambda i: (0, 0)),
            pl.BlockSpec((1, O), lambda i: (0, 0)),
        ],
        out_specs=pl.BlockSpec((TB, O), lambda i: (i, 0)),
        out_shape=jax.ShapeDtypeStruct((B, O), jnp.float32),
        compiler_params=pltpu.CompilerParams(
            dimension_semantics=("parallel",)),
    )(x, W1, b1.reshape(1, H), W2, b2.reshape(1, O))
    return out


# speedup vs baseline: 651.0796x; 1.1726x over previous
import jax
import jax.numpy as jnp
import numpy as np
from jax.experimental import pallas as pl
from jax.experimental.pallas import tpu as pltpu

_L = 1000


def _build_candidates():
    one = np.float32(1.0)
    thr = np.float32(0.5)
    traj = np.empty(_L, np.float32)
    traj[0] = np.float32(0.5)
    for i in range(1, _L):
        xp = traj[i - 1]
        traj[i] = xp / thr if xp < thr else (one - xp) / (one - thr)
    z = np.zeros(1, np.float32)
    cs_gt = np.concatenate(
        [z, np.cumsum((traj > 0.5).astype(np.float32), dtype=np.float32)])
    cs_sq = np.concatenate([z, np.cumsum(traj * traj, dtype=np.float32)])
    cs_ent = np.concatenate(
        [z, np.cumsum(traj * np.log2(traj + np.float32(1e-10)),
                      dtype=np.float32)])
    _, first = np.unique(traj, return_index=True)
    cands = []
    for i in np.sort(first):
        i = int(i)
        cands.append((float(traj[i]), float(np.float32(i)),
                      float(cs_gt[i]), float(cs_sq[i]), float(cs_ent[i])))
    return cands


_CANDS = _build_candidates()


def _fused_kernel(x_ref, w1_ref, b1_ref, w2_ref, b2_ref, o_ref):
    v = x_ref[...]
    u0, i0, g0, s0, e0 = _CANDS[0]
    best = jnp.abs(v - u0)
    f_idx = jnp.full_like(v, i0)
    f_gt = jnp.full_like(v, g0)
    f_sq = jnp.full_like(v, s0)
    f_ent = jnp.full_like(v, e0)
    for (u, i, g, s, e) in _CANDS[1:]:
        d = jnp.abs(v - u)
        m = d < best
        best = jnp.where(m, d, best)
        f_idx = jnp.where(m, i, f_idx)
        f_gt = jnp.where(m, g, f_gt)
        f_sq = jnp.where(m, s, f_sq)
        f_ent = jnp.where(m, e, f_ent)
    TB, D = v.shape
    blocked = jnp.concatenate([f_gt, f_sq, f_idx, -f_ent], axis=1)
    r = jax.lax.broadcasted_iota(jnp.int32, (4 * D, 4 * D), 0)
    j = jax.lax.broadcasted_iota(jnp.int32, (4 * D, 4 * D), 1)
    perm = (j == 4 * (r % D) + r // D).astype(jnp.float32)
    feats0 = jax.lax.dot_general(blocked, perm, (((1,), (0,)), ((), ())),
                                 preferred_element_type=jnp.float32)
    rd = jax.lax.broadcasted_iota(jnp.int32, (D, 4 * D), 0)
    jd = jax.lax.broadcasted_iota(jnp.int32, (D, 4 * D), 1)
    sel0 = (jd == 4 * rd).astype(jnp.float32)
    idx0 = jax.lax.dot_general(f_idx, sel0, (((1,), (0,)), ((), ())),
                               preferred_element_type=jnp.float32)
    lane_c = jax.lax.broadcasted_iota(jnp.int32, (TB, 4 * D), 1) % 4
    feats = jnp.where(lane_c == 0, feats0 / idx0, feats0)
    h = jax.lax.dot_general(feats, w1_ref[...], (((1,), (1,)), ((), ())),
                            preferred_element_type=jnp.float32)
    h = jnp.maximum(h + b1_ref[...], 0.0)
    o_ref[...] = (jax.lax.dot_general(h, w2_ref[...], (((1,), (1,)), ((), ())),
                                      preferred_element_type=jnp.float32)
                  + b2_ref[...])


def kernel(x, W1, b1, W2, b2):
    B, D = x.shape
    H = W1.shape[0]
    O = W2.shape[0]
    TB = 2048
    out = pl.pallas_call(
        _fused_kernel,
        grid=(B // TB,),
        in_specs=[
            pl.BlockSpec((TB, D), lambda i: (i, 0)),
            pl.BlockSpec((H, 4 * D), lambda i: (0, 0)),
            pl.BlockSpec((1, H), lambda i: (0, 0)),
            pl.BlockSpec((O, H), lambda i: (0, 0)),
            pl.BlockSpec((1, O), lambda i: (0, 0)),
        ],
        out_specs=pl.BlockSpec((TB, O), lambda i: (i, 0)),
        out_shape=jax.ShapeDtypeStruct((B, O), jnp.float32),
        compiler_params=pltpu.CompilerParams(
            dimension_semantics=("parallel",)),
    )(x, W1, b1.reshape(1, H), W2, b2.reshape(1, O))
    return out
